# baseline (speedup 1.0000x reference)
import jax
import jax.numpy as jnp
import numpy as np
from jax.experimental import pallas as pl
from jax.experimental.pallas import tpu as pltpu

B, S, D = 4, 8192, 768
BLK = 512
NJ = S // BLK
NR = NJ + 1
K = S // 8

_TOP = np.uint32(0x80000000)


def _mod_kernel(x_ref, wr_ref, W_ref, bb_ref, o_ref, xring, keyring,
                keys_scr, kthr_scr):
    q = pl.program_id(0)
    j = pl.program_id(1)
    t = q * NJ + j

    @pl.when(q < B)
    def _producer():
        slot = jax.lax.rem(t, NR)
        xb = x_ref[0]
        xring[slot] = xb
        wv = jax.lax.dot_general(
            xb, wr_ref[...], (((1,), (1,)), ((), ())),
            preferred_element_type=jnp.float32)[:, :1]
        u = jax.lax.bitcast_convert_type(wv, jnp.uint32)
        key = jnp.where((u & _TOP) != 0, ~u, u | _TOP)
        keyring[slot] = key
        keys_scr[j] = key.T

        @pl.when(j == NJ - 1)
        def _find_threshold():
            keys = keys_scr[...]

            def body(_, lohi):
                lo, hi = lohi
                span = hi - lo
                mid = lo + (span >> 1) + (span & np.uint32(1))
                cnt = jnp.sum((keys >= mid).astype(jnp.int32), axis=2,
                              keepdims=True)
                cnt = jnp.sum(cnt, axis=0, keepdims=True)
                sel = cnt >= K
                return (jnp.where(sel, mid, lo),
                        jnp.where(sel, hi, mid - np.uint32(1)))

            lo0 = jnp.zeros((1, 1, 1), jnp.uint32)
            hi0 = jnp.full((1, 1, 1), 0xFFFFFFFF, jnp.uint32)
            lo, _ = jax.lax.fori_loop(0, 32, body, (lo0, hi0))
            kthr_scr[jax.lax.rem(q, 2)] = jnp.broadcast_to(lo[0], (BLK, 1))

    @pl.when(q > 0)
    def _consumer():
        cslot = jax.lax.rem(t - NJ, NR)
        xb = xring[cslot]
        mask = keyring[cslot] > kthr_scr[jax.lax.rem(q - 1, 2)]
        y = jnp.dot(xb.astype(jnp.bfloat16), W_ref[...],
                    preferred_element_type=jnp.float32) + bb_ref[...]
        o_ref[0] = jnp.where(mask, y, xb)


def kernel(x, W_router, b_router, W_block, b_block):
    del b_router
    wr = jnp.zeros((128, D), jnp.float32).at[0].set(W_router[:, 0])
    W16 = W_block.astype(jnp.bfloat16)
    bb = b_block.reshape(1, D)

    def x_map(q, j):
        lt = (q < B).astype(jnp.int32)
        return (jnp.minimum(q, B - 1), j * lt + (NJ - 1) * (1 - lt), 0)

    def o_map(q, j):
        gt = (q > 0).astype(jnp.int32)
        return (jnp.maximum(q - 1, 0), j * gt, 0)

    out = pl.pallas_call(
        _mod_kernel,
        grid=(B + 1, NJ),
        in_specs=[
            pl.BlockSpec((1, BLK, D), x_map),
            pl.BlockSpec((128, D), lambda q, j: (0, 0)),
            pl.BlockSpec((D, D), lambda q, j: (0, 0)),
            pl.BlockSpec((1, D), lambda q, j: (0, 0)),
        ],
        out_specs=pl.BlockSpec((1, BLK, D), o_map),
        out_shape=jax.ShapeDtypeStruct((B, S, D), jnp.float32),
        scratch_shapes=[
            pltpu.VMEM((NR, BLK, D), jnp.float32),
            pltpu.VMEM((NR, BLK, 1), jnp.uint32),
            pltpu.VMEM((NJ, 1, BLK), jnp.uint32),
            pltpu.VMEM((2, BLK, 1), jnp.uint32),
        ],
    )(x, wr, W16, bb)
    return out


# speedup vs baseline: 1.2461x; 1.2461x over previous
import jax
import jax.numpy as jnp
import numpy as np
from jax.experimental import pallas as pl
from jax.experimental.pallas import tpu as pltpu

B, S, D = 4, 8192, 768
BLK = 1024
NJ = S // BLK
NR = NJ + 1
K = S // 8

_TOP = np.uint32(0x80000000)


def _mod_kernel(x_ref, wr_ref, W_ref, bb_ref, o_ref, xring, keyring,
                keys_scr, kthr_scr):
    q = pl.program_id(0)
    j = pl.program_id(1)
    t = q * NJ + j

    @pl.when(q < B)
    def _producer():
        slot = jax.lax.rem(t, NR)
        xb = x_ref[0]
        xring[slot] = xb
        wv = jax.lax.dot_general(
            xb, wr_ref[...], (((1,), (1,)), ((), ())),
            preferred_element_type=jnp.float32)[:, :1]
        u = jax.lax.bitcast_convert_type(wv, jnp.uint32)
        key = jnp.where((u & _TOP) != 0, ~u, u | _TOP)
        keyring[slot] = key
        keys_scr[j] = key.T

        @pl.when(j == NJ - 1)
        def _find_threshold():
            keys = keys_scr[...]

            def body(_, lohi):
                lo, hi = lohi
                span = hi - lo
                mid = lo + (span >> 1) + (span & np.uint32(1))
                cnt = jnp.sum((keys >= mid).astype(jnp.int32), axis=2,
                              keepdims=True)
                cnt = jnp.sum(cnt, axis=0, keepdims=True)
                sel = cnt >= K
                return (jnp.where(sel, mid, lo),
                        jnp.where(sel, hi, mid - np.uint32(1)))

            lo0 = jnp.zeros((1, 1, 1), jnp.uint32)
            hi0 = jnp.full((1, 1, 1), 0xFFFFFFFF, jnp.uint32)
            lo, _ = jax.lax.fori_loop(0, 32, body, (lo0, hi0))
            kthr_scr[jax.lax.rem(q, 2)] = jnp.broadcast_to(lo[0], (BLK, 1))

    @pl.when(q > 0)
    def _consumer():
        cslot = jax.lax.rem(t - NJ, NR)
        xb = xring[cslot]
        mask = keyring[cslot] > kthr_scr[jax.lax.rem(q - 1, 2)]
        y = jnp.dot(xb.astype(jnp.bfloat16), W_ref[...],
                    preferred_element_type=jnp.float32) + bb_ref[...]
        o_ref[0] = jnp.where(mask, y, xb)


def kernel(x, W_router, b_router, W_block, b_block):
    del b_router
    wr = jnp.zeros((8, D), jnp.float32).at[0].set(W_router[:, 0])
    W16 = W_block.astype(jnp.bfloat16)
    bb = b_block.reshape(1, D)

    def x_map(q, j):
        lt = (q < B).astype(jnp.int32)
        return (jnp.minimum(q, B - 1), j * lt + (NJ - 1) * (1 - lt), 0)

    def o_map(q, j):
        gt = (q > 0).astype(jnp.int32)
        return (jnp.maximum(q - 1, 0), j * gt, 0)

    out = pl.pallas_call(
        _mod_kernel,
        grid=(B + 1, NJ),
        in_specs=[
            pl.BlockSpec((1, BLK, D), x_map),
            pl.BlockSpec((8, D), lambda q, j: (0, 0)),
            pl.BlockSpec((D, D), lambda q, j: (0, 0)),
            pl.BlockSpec((1, D), lambda q, j: (0, 0)),
        ],
        out_specs=pl.BlockSpec((1, BLK, D), o_map),
        out_shape=jax.ShapeDtypeStruct((B, S, D), jnp.float32),
        scratch_shapes=[
            pltpu.VMEM((NR, BLK, D), jnp.float32),
            pltpu.VMEM((NR, BLK, 1), jnp.uint32),
            pltpu.VMEM((NJ, 1, BLK), jnp.uint32),
            pltpu.VMEM((2, BLK, 1), jnp.uint32),
        ],
    )(x, wr, W16, bb)
    return out
